# baseline (speedup 1.0000x reference)
import functools

import jax
import jax.numpy as jnp
from jax import lax
from jax.experimental import pallas as pl
from jax.experimental.pallas import tpu as pltpu
from jax.experimental.pallas import tpu_sc as plsc

NEMB = 64
NVEC = 32
B = 256
NODES_PER = 20
N = B * NODES_PER
EDGES_PER = 38
E = B * EDGES_PER
STEMS_PER = 4
S = B * STEMS_PER
NUM_CONV_STEPS = 6
OUT_PER_STEM = 105
OUT_PER_MOL = 1
NUM_BLOCKS = 105
NUM_STEM_TYPES = 73

C = 16
G = B // C
CN = C * NODES_PER
CE = C * EDGES_PER
CS = C * STEMS_PER

NC = 2
NS = 16
NW = NC * NS
EPW = E // NW
ECH = 76
NCHUNK = EPW // ECH
HALF = N // NC
RPT = HALF // NS

NP = 128
RB = 640
GR = N // RB

_F32 = jnp.float32


def _lrelu(v):
    return jnp.where(v >= 0, v, 0.01 * v)


def _prologue_body(x_ref, ea0_ref, ea1_ref, dstc_ref, dstr_ref, vec_ref,
                   blockemb_ref, bondemb_ref,
                   b2e_W1T_ref, b2e_b1_ref, b2e_W2T_ref, b2e_b2_ref,
                   out_ref, a_ref, bs_ref):
    pid = pl.program_id(0)
    base = pid * CN

    def dot(p, q):
        return jnp.dot(p, q, preferred_element_type=_F32)

    xv = x_ref[0]
    oh_x = (xv == lax.broadcasted_iota(
        jnp.int32, (CN, NUM_BLOCKS + 1), 1)).astype(_F32)
    xe = dot(oh_x, blockemb_ref[...])

    rep = (lax.broadcasted_iota(jnp.int32, (CN, C), 0) // NODES_PER ==
           lax.broadcasted_iota(jnp.int32, (CN, C), 1)).astype(_F32)
    bvec = dot(rep, vec_ref[...])

    inp = jnp.concatenate([xe, bvec], axis=1)
    hh = _lrelu(dot(inp, b2e_W1T_ref[...]) + b2e_b1_ref[...])
    out64 = dot(hh, b2e_W2T_ref[...]) + b2e_b2_ref[...]
    out_ref[...] = jnp.concatenate(
        [out64, jnp.zeros((CN, NP - NEMB), _F32)], axis=1)

    oh_a = (ea0_ref[0] == lax.broadcasted_iota(
        jnp.int32, (CE, NUM_STEM_TYPES), 1)).astype(_F32)
    oh_b = (ea1_ref[0] == lax.broadcasted_iota(
        jnp.int32, (CE, NUM_STEM_TYPES), 1)).astype(_F32)
    a_ref[...] = dot(oh_a, bondemb_ref[...])
    ea_b = dot(oh_b, bondemb_ref[...])

    dstl_r = dstr_ref[0, 0, :] - base
    g_dst_t = (dstl_r[None, :] == lax.broadcasted_iota(
        jnp.int32, (CN, CE), 0)).astype(_F32)
    deg = jnp.sum(g_dst_t, axis=1, keepdims=True)
    inv_denom = 1.0 / jnp.maximum(deg, 1.0)
    dstl_c = dstc_ref[0] - base
    g_dst = (dstl_c == lax.broadcasted_iota(
        jnp.int32, (CE, CN), 1)).astype(_F32)
    inv_e = dot(g_dst, inv_denom)
    bs_ref[...] = ea_b * inv_e


def _sc_step_body(out_hbm, srcw_hbm, dstw_hbm, a_hbm, b_hbm, zeros_hbm,
                  agg_hbm,
                  srcw_v, dstw_v, rows_v, a_v, b_v, agg_sh, sem):
    c = lax.axis_index("c")
    s = lax.axis_index("s")
    w = c * NS + s
    rbase = s * RPT

    pltpu.sync_copy(srcw_hbm.at[w], srcw_v)
    pltpu.sync_copy(dstw_hbm.at[w], dstw_v)
    pltpu.sync_copy(zeros_hbm.at[pl.ds(rbase, RPT)],
                    agg_sh.at[pl.ds(rbase, RPT)])
    plsc.subcore_barrier()

    dnums = lax.GatherDimensionNumbers(
        offset_dims=(), collapsed_slice_dims=(0,), start_index_map=(0,))
    lane = lax.broadcasted_iota(jnp.int32, (16,), 0)
    bfly = [(lane ^ sh).reshape(16, 1) for sh in (1, 2, 4, 8)]

    for j in range(NCHUNK):
        pltpu.sync_copy(a_hbm.at[w, j], a_v)
        pltpu.sync_copy(b_hbm.at[w, j], b_v)
        pltpu.async_copy(out_hbm.at[srcw_v.at[j]], rows_v, sem).wait()

        def edge_body(e, carry):
            acc = rows_v[e, pl.ds(0, 16)] * a_v[e, pl.ds(0, 16)]
            for q in range(1, NEMB // 16):
                acc = acc + (rows_v[e, pl.ds(q * 16, 16)] *
                             a_v[e, pl.ds(q * 16, 16)])
            for idx in bfly:
                acc = acc + lax.gather(
                    acc, idx, dnums, (1,),
                    mode=lax.GatherScatterMode.PROMISE_IN_BOUNDS)
            for q in range(NEMB // 16):
                rows_v[e, pl.ds(q * 16, 16)] = acc * b_v[e, pl.ds(q * 16, 16)]
            return carry

        lax.fori_loop(0, ECH, edge_body, 0)
        pltpu.sync_copy(rows_v, agg_sh.at[dstw_v.at[j]], add=True)
    plsc.subcore_barrier()
    pltpu.sync_copy(agg_sh.at[pl.ds(rbase, RPT)],
                    agg_hbm.at[pl.ds(c * HALF + rbase, RPT)])


def _gru_body(agg_ref, h_ref, conv_root_ref, conv_bias_ref,
              gru_WihT_ref, gru_bih_ref, gru_WhhT_ref, gru_bhh_ref,
              hout_ref):
    def dot(p, q):
        return jnp.dot(p, q, preferred_element_type=_F32)

    h = h_ref[:, :NEMB]
    m = _lrelu(agg_ref[:, :NEMB] + dot(h, conv_root_ref[...]) + conv_bias_ref[...])
    gi = dot(m, gru_WihT_ref[...]) + gru_bih_ref[...]
    gh = dot(h, gru_WhhT_ref[...]) + gru_bhh_ref[...]
    r = jax.nn.sigmoid(gi[:, :NEMB] + gh[:, :NEMB])
    z = jax.nn.sigmoid(gi[:, NEMB:2 * NEMB] + gh[:, NEMB:2 * NEMB])
    n = jnp.tanh(gi[:, 2 * NEMB:] + r * gh[:, 2 * NEMB:])
    hout_ref[...] = jnp.concatenate(
        [(1.0 - z) * n + z * h, jnp.zeros((RB, NP - NEMB), _F32)], axis=1)


def _heads_body(h_ref, stt_ref, stem0_ref, stememb_ref,
                s2p_W1T_ref, s2p_b1_ref, s2p_W2T_ref, s2p_b2_ref,
                s2p_W3T_ref, s2p_b3_ref,
                g2p_W1T_ref, g2p_b1_ref, g2p_W2T_ref, g2p_b2_ref,
                stem_out_ref, mol_out_ref):
    def dot(p, q):
        return jnp.dot(p, q, preferred_element_type=_F32)

    out = h_ref[:, :NEMB]
    stt = stt_ref[0]
    oh_st = (stt == lax.broadcasted_iota(
        jnp.int32, (CS, NUM_STEM_TYPES + 1), 1)).astype(_F32)
    st = dot(oh_st, stememb_ref[...])

    stem0 = stem0_ref[0]
    sidx = (lax.broadcasted_iota(jnp.int32, (CS, 1), 0) // STEMS_PER
            ) * NODES_PER + stem0
    sel = (sidx == lax.broadcasted_iota(
        jnp.int32, (CS, CN), 1)).astype(_F32)
    stem_x = dot(sel, out)

    cat = jnp.concatenate([stem_x, st], axis=1)
    sh = _lrelu(dot(cat, s2p_W1T_ref[...]) + s2p_b1_ref[...])
    sh = _lrelu(dot(sh, s2p_W2T_ref[...]) + s2p_b2_ref[...])
    stem_out_ref[...] = dot(sh, s2p_W3T_ref[...]) + s2p_b3_ref[...]

    pool = (lax.broadcasted_iota(jnp.int32, (C, CN), 0) ==
            lax.broadcasted_iota(jnp.int32, (C, CN), 1) // NODES_PER
            ).astype(_F32) * (1.0 / NODES_PER)
    gmean = dot(pool, out)
    mh = _lrelu(dot(gmean, g2p_W1T_ref[...]) + g2p_b1_ref[...])
    mol_out_ref[...] = dot(mh, g2p_W2T_ref[...]) + g2p_b2_ref[...]


@jax.jit
def _run(params, vec_data, x, stemtypes, edge_attr, edge_index, stems):
    p = params
    i32 = jnp.int32

    def row(v):
        return v.reshape(1, -1).astype(_F32)

    x3 = x.astype(i32).reshape(G, CN, 1)
    stt3 = stemtypes.astype(i32).reshape(G, CS, 1)
    ea0 = edge_attr[:, 0].astype(i32).reshape(G, CE, 1)
    ea1 = edge_attr[:, 1].astype(i32).reshape(G, CE, 1)
    src = edge_index[0].astype(i32)
    dst = edge_index[1].astype(i32)
    dstc = dst.reshape(G, CE, 1)
    dstr = dst.reshape(G, 1, CE)
    stem03 = stems[:, 0].astype(i32).reshape(G, CS, 1)
    vec = vec_data.astype(_F32)

    col_spec = lambda L: pl.BlockSpec((1, L, 1), lambda i: (i, 0, 0))
    row_spec = lambda L: pl.BlockSpec((1, 1, L), lambda i: (i, 0, 0))

    def w_spec(w):
        return pl.BlockSpec(w.shape, lambda i: (0, 0))

    pro_w = (p['blockemb'].astype(_F32), p['bondemb'].astype(_F32),
             p['b2e_W1'].T.astype(_F32), row(p['b2e_b1']),
             p['b2e_W2'].T.astype(_F32), row(p['b2e_b2']))
    out0, ea_a, ea_bs = pl.pallas_call(
        _prologue_body,
        grid=(G,),
        in_specs=[
            col_spec(CN), col_spec(CE), col_spec(CE), col_spec(CE),
            row_spec(CE),
            pl.BlockSpec((C, NVEC), lambda i: (i, 0)),
            *[w_spec(w) for w in pro_w],
        ],
        out_specs=[
            pl.BlockSpec((CN, NP), lambda i: (i, 0)),
            pl.BlockSpec((CE, NEMB), lambda i: (i, 0)),
            pl.BlockSpec((CE, NEMB), lambda i: (i, 0)),
        ],
        out_shape=[
            jax.ShapeDtypeStruct((N, NP), _F32),
            jax.ShapeDtypeStruct((E, NEMB), _F32),
            jax.ShapeDtypeStruct((E, NEMB), _F32),
        ],
    )(x3, ea0, ea1, dstc, dstr, vec, *pro_w)

    srcw = src.reshape(NW, NCHUNK, ECH)
    dstw = (dst % HALF).reshape(NW, NCHUNK, ECH)
    ea_a4 = ea_a.reshape(NW, NCHUNK, ECH, NEMB)
    ea_bs4 = ea_bs.reshape(NW, NCHUNK, ECH, NEMB)
    zeros = jnp.zeros((HALF, NP), _F32)

    sc_step = functools.partial(
        pl.kernel,
        mesh=plsc.VectorSubcoreMesh(core_axis_name="c", subcore_axis_name="s"),
        out_type=jax.ShapeDtypeStruct((N, NP), _F32),
        scratch_types=[
            pltpu.VMEM((NCHUNK, ECH), i32),
            pltpu.VMEM((NCHUNK, ECH), i32),
            pltpu.VMEM((ECH, NP), _F32),
            pltpu.VMEM((ECH, NEMB), _F32),
            pltpu.VMEM((ECH, NEMB), _F32),
            pltpu.VMEM_SHARED((HALF, NP), _F32),
            pltpu.SemaphoreType.DMA,
        ],
    )(_sc_step_body)

    gru_w = (p['conv_root'].astype(_F32), row(p['conv_bias']),
             p['gru_Wih'].T.astype(_F32), row(p['gru_bih']),
             p['gru_Whh'].T.astype(_F32), row(p['gru_bhh']))

    def gru_step(agg, h):
        return pl.pallas_call(
            _gru_body,
            grid=(GR,),
            in_specs=[
                pl.BlockSpec((RB, NP), lambda i: (i, 0)),
                pl.BlockSpec((RB, NP), lambda i: (i, 0)),
                *[w_spec(w) for w in gru_w],
            ],
            out_specs=pl.BlockSpec((RB, NP), lambda i: (i, 0)),
            out_shape=jax.ShapeDtypeStruct((N, NP), _F32),
        )(agg, h, *gru_w)

    h = out0
    for _ in range(NUM_CONV_STEPS):
        agg = sc_step(h, srcw, dstw, ea_a4, ea_bs4, zeros)
        h = gru_step(agg, h)

    head_w = (p['stememb'].astype(_F32),
              p['s2p_W1'].T.astype(_F32), row(p['s2p_b1']),
              p['s2p_W2'].T.astype(_F32), row(p['s2p_b2']),
              p['s2p_W3'].T.astype(_F32), row(p['s2p_b3']),
              p['g2p_W1'].T.astype(_F32), row(p['g2p_b1']),
              p['g2p_W2'].T.astype(_F32), row(p['g2p_b2']))
    stem_preds, mol_preds = pl.pallas_call(
        _heads_body,
        grid=(G,),
        in_specs=[
            pl.BlockSpec((CN, NP), lambda i: (i, 0)),
            col_spec(CS), col_spec(CS),
            *[w_spec(w) for w in head_w],
        ],
        out_specs=[
            pl.BlockSpec((CS, OUT_PER_STEM), lambda i: (i, 0)),
            pl.BlockSpec((C, OUT_PER_MOL), lambda i: (i, 0)),
        ],
        out_shape=[
            jax.ShapeDtypeStruct((S, OUT_PER_STEM), _F32),
            jax.ShapeDtypeStruct((B, OUT_PER_MOL), _F32),
        ],
    )(h, stt3, stem03, *head_w)
    return stem_preds, mol_preds


def kernel(params, vec_data, x, stemtypes, edge_attr, edge_index, batch,
           stems_batch, stems, slices_x):
    return _run(params, vec_data, x, stemtypes, edge_attr, edge_index, stems)


# speedup vs baseline: 1.2074x; 1.2074x over previous
import functools

import jax
import jax.numpy as jnp
from jax import lax
from jax.experimental import pallas as pl
from jax.experimental.pallas import tpu as pltpu
from jax.experimental.pallas import tpu_sc as plsc

NEMB = 64
NVEC = 32
B = 256
NODES_PER = 20
N = B * NODES_PER
EDGES_PER = 38
E = B * EDGES_PER
STEMS_PER = 4
S = B * STEMS_PER
NUM_CONV_STEPS = 6
OUT_PER_STEM = 105
OUT_PER_MOL = 1
NUM_BLOCKS = 105
NUM_STEM_TYPES = 73

C = 16
G = B // C
CN = C * NODES_PER
CE = C * EDGES_PER
CS = C * STEMS_PER

NC = 2
NS = 16
NW = NC * NS
EPW = E // NW
ECH = 76
NCHUNK = EPW // ECH
HALF = N // NC
RPT = HALF // NS

NP = 128
RB = 640
GR = N // RB

_F32 = jnp.float32


def _lrelu(v):
    return jnp.where(v >= 0, v, 0.01 * v)


def _prologue_body(x_ref, ea0_ref, ea1_ref, dstc_ref, dstr_ref, vec_ref,
                   blockemb_ref, bondemb_ref,
                   b2e_W1T_ref, b2e_b1_ref, b2e_W2T_ref, b2e_b2_ref,
                   out_ref, ab_ref):
    pid = pl.program_id(0)
    base = pid * CN

    def dot(p, q):
        return jnp.dot(p, q, preferred_element_type=_F32)

    xv = x_ref[0]
    oh_x = (xv == lax.broadcasted_iota(
        jnp.int32, (CN, NUM_BLOCKS + 1), 1)).astype(_F32)
    xe = dot(oh_x, blockemb_ref[...])

    rep = (lax.broadcasted_iota(jnp.int32, (CN, C), 0) // NODES_PER ==
           lax.broadcasted_iota(jnp.int32, (CN, C), 1)).astype(_F32)
    bvec = dot(rep, vec_ref[...])

    inp = jnp.concatenate([xe, bvec], axis=1)
    hh = _lrelu(dot(inp, b2e_W1T_ref[...]) + b2e_b1_ref[...])
    out64 = dot(hh, b2e_W2T_ref[...]) + b2e_b2_ref[...]
    out_ref[...] = jnp.concatenate(
        [out64, jnp.zeros((CN, NP - NEMB), _F32)], axis=1)

    oh_a = (ea0_ref[0] == lax.broadcasted_iota(
        jnp.int32, (CE, NUM_STEM_TYPES), 1)).astype(_F32)
    oh_b = (ea1_ref[0] == lax.broadcasted_iota(
        jnp.int32, (CE, NUM_STEM_TYPES), 1)).astype(_F32)
    ea_a = dot(oh_a, bondemb_ref[...])
    ea_b = dot(oh_b, bondemb_ref[...])

    dstl_r = dstr_ref[0, 0, :] - base
    g_dst_t = (dstl_r[None, :] == lax.broadcasted_iota(
        jnp.int32, (CN, CE), 0)).astype(_F32)
    deg = jnp.sum(g_dst_t, axis=1, keepdims=True)
    inv_denom = 1.0 / jnp.maximum(deg, 1.0)
    dstl_c = dstc_ref[0] - base
    g_dst = (dstl_c == lax.broadcasted_iota(
        jnp.int32, (CE, CN), 1)).astype(_F32)
    inv_e = dot(g_dst, inv_denom)
    ab_ref[...] = jnp.concatenate([ea_a, ea_b * inv_e], axis=1)


def _sc_step_body(out_hbm, srcw_hbm, dstw_hbm, ab_hbm, agg_hbm,
                  srcw_v, dstw_v, rows_v, ab_v, acc_v, sem):
    c = lax.axis_index("c")
    s = lax.axis_index("s")
    w = c * NS + s

    pltpu.sync_copy(srcw_hbm.at[w], srcw_v)
    pltpu.sync_copy(dstw_hbm.at[w], dstw_v)
    cps = [pltpu.async_copy(ab_hbm.at[w], ab_v, sem)]
    for j in range(NCHUNK):
        cps.append(pltpu.async_copy(out_hbm.at[srcw_v.at[j]],
                                    rows_v.at[pl.ds(j * ECH, ECH)], sem))

    zv = jnp.zeros((16,), _F32)

    def zero_body(r, carry):
        for q in range(NEMB // 16):
            acc_v[r, pl.ds(q * 16, 16)] = zv
        return carry

    lax.fori_loop(0, RPT, zero_body, 0)
    for cp in cps:
        cp.wait()

    dnums = lax.GatherDimensionNumbers(
        offset_dims=(), collapsed_slice_dims=(0,), start_index_map=(0,))
    lane = lax.broadcasted_iota(jnp.int32, (16,), 0)
    bfly = [(lane ^ sh).reshape(16, 1) for sh in (1, 2, 4, 8)]

    def group_body(g, carry):
        dv = dstw_v[g]
        for l in range(16):
            e = g * 16 + l
            acc = rows_v[e, pl.ds(0, 16)] * ab_v[e, pl.ds(0, 16)]
            for q in range(1, NEMB // 16):
                acc = acc + (rows_v[e, pl.ds(q * 16, 16)] *
                             ab_v[e, pl.ds(q * 16, 16)])
            for idx in bfly:
                acc = acc + lax.gather(
                    acc, idx, dnums, (1,),
                    mode=lax.GatherScatterMode.PROMISE_IN_BOUNDS)
            d = dv[l]
            for q in range(NEMB // 16):
                acc_v[d, pl.ds(q * 16, 16)] = (
                    acc_v[d, pl.ds(q * 16, 16)] +
                    acc * ab_v[e, pl.ds(NEMB + q * 16, 16)])
        return carry

    lax.fori_loop(0, EPW // 16, group_body, 0)

    pltpu.sync_copy(acc_v, agg_hbm.at[pl.ds(w * RPT, RPT)])


def _gru_body(agg_ref, h_ref, conv_root_ref, conv_bias_ref,
              gru_WihT_ref, gru_bih_ref, gru_WhhT_ref, gru_bhh_ref,
              hout_ref):
    def dot(p, q):
        return jnp.dot(p, q, preferred_element_type=_F32)

    h = h_ref[:, :NEMB]
    m = _lrelu(agg_ref[...] + dot(h, conv_root_ref[...]) + conv_bias_ref[...])
    gi = dot(m, gru_WihT_ref[...]) + gru_bih_ref[...]
    gh = dot(h, gru_WhhT_ref[...]) + gru_bhh_ref[...]
    r = jax.nn.sigmoid(gi[:, :NEMB] + gh[:, :NEMB])
    z = jax.nn.sigmoid(gi[:, NEMB:2 * NEMB] + gh[:, NEMB:2 * NEMB])
    n = jnp.tanh(gi[:, 2 * NEMB:] + r * gh[:, 2 * NEMB:])
    hout_ref[...] = jnp.concatenate(
        [(1.0 - z) * n + z * h, jnp.zeros((RB, NP - NEMB), _F32)], axis=1)


def _heads_body(h_ref, stt_ref, stem0_ref, stememb_ref,
                s2p_W1T_ref, s2p_b1_ref, s2p_W2T_ref, s2p_b2_ref,
                s2p_W3T_ref, s2p_b3_ref,
                g2p_W1T_ref, g2p_b1_ref, g2p_W2T_ref, g2p_b2_ref,
                stem_out_ref, mol_out_ref):
    def dot(p, q):
        return jnp.dot(p, q, preferred_element_type=_F32)

    out = h_ref[:, :NEMB]
    stt = stt_ref[0]
    oh_st = (stt == lax.broadcasted_iota(
        jnp.int32, (CS, NUM_STEM_TYPES + 1), 1)).astype(_F32)
    st = dot(oh_st, stememb_ref[...])

    stem0 = stem0_ref[0]
    sidx = (lax.broadcasted_iota(jnp.int32, (CS, 1), 0) // STEMS_PER
            ) * NODES_PER + stem0
    sel = (sidx == lax.broadcasted_iota(
        jnp.int32, (CS, CN), 1)).astype(_F32)
    stem_x = dot(sel, out)

    cat = jnp.concatenate([stem_x, st], axis=1)
    sh = _lrelu(dot(cat, s2p_W1T_ref[...]) + s2p_b1_ref[...])
    sh = _lrelu(dot(sh, s2p_W2T_ref[...]) + s2p_b2_ref[...])
    stem_out_ref[...] = dot(sh, s2p_W3T_ref[...]) + s2p_b3_ref[...]

    pool = (lax.broadcasted_iota(jnp.int32, (C, CN), 0) ==
            lax.broadcasted_iota(jnp.int32, (C, CN), 1) // NODES_PER
            ).astype(_F32) * (1.0 / NODES_PER)
    gmean = dot(pool, out)
    mh = _lrelu(dot(gmean, g2p_W1T_ref[...]) + g2p_b1_ref[...])
    mol_out_ref[...] = dot(mh, g2p_W2T_ref[...]) + g2p_b2_ref[...]


@jax.jit
def _run(params, vec_data, x, stemtypes, edge_attr, edge_index, stems):
    p = params
    i32 = jnp.int32

    def row(v):
        return v.reshape(1, -1).astype(_F32)

    x3 = x.astype(i32).reshape(G, CN, 1)
    stt3 = stemtypes.astype(i32).reshape(G, CS, 1)
    ea0 = edge_attr[:, 0].astype(i32).reshape(G, CE, 1)
    ea1 = edge_attr[:, 1].astype(i32).reshape(G, CE, 1)
    src = edge_index[0].astype(i32)
    dst = edge_index[1].astype(i32)
    dstc = dst.reshape(G, CE, 1)
    dstr = dst.reshape(G, 1, CE)
    stem03 = stems[:, 0].astype(i32).reshape(G, CS, 1)
    vec = vec_data.astype(_F32)

    col_spec = lambda L: pl.BlockSpec((1, L, 1), lambda i: (i, 0, 0))
    row_spec = lambda L: pl.BlockSpec((1, 1, L), lambda i: (i, 0, 0))

    def w_spec(w):
        return pl.BlockSpec(w.shape, lambda i: (0, 0))

    pro_w = (p['blockemb'].astype(_F32), p['bondemb'].astype(_F32),
             p['b2e_W1'].T.astype(_F32), row(p['b2e_b1']),
             p['b2e_W2'].T.astype(_F32), row(p['b2e_b2']))
    out0, ea_ab = pl.pallas_call(
        _prologue_body,
        grid=(G,),
        in_specs=[
            col_spec(CN), col_spec(CE), col_spec(CE), col_spec(CE),
            row_spec(CE),
            pl.BlockSpec((C, NVEC), lambda i: (i, 0)),
            *[w_spec(w) for w in pro_w],
        ],
        out_specs=[
            pl.BlockSpec((CN, NP), lambda i: (i, 0)),
            pl.BlockSpec((CE, NP), lambda i: (i, 0)),
        ],
        out_shape=[
            jax.ShapeDtypeStruct((N, NP), _F32),
            jax.ShapeDtypeStruct((E, NP), _F32),
        ],
    )(x3, ea0, ea1, dstc, dstr, vec, *pro_w)

    srcw = src.reshape(NW, NCHUNK, ECH)
    dstw = (dst % RPT).reshape(NW, EPW // 16, 16)
    ab3 = ea_ab.reshape(NW, EPW, NP)

    sc_step = functools.partial(
        pl.kernel,
        mesh=plsc.VectorSubcoreMesh(core_axis_name="c", subcore_axis_name="s"),
        out_type=jax.ShapeDtypeStruct((N, NEMB), _F32),
        scratch_types=[
            pltpu.VMEM((NCHUNK, ECH), i32),
            pltpu.VMEM((EPW // 16, 16), i32),
            pltpu.VMEM((EPW, NP), _F32),
            pltpu.VMEM((EPW, NP), _F32),
            pltpu.VMEM((RPT, NEMB), _F32),
            pltpu.SemaphoreType.DMA,
        ],
    )(_sc_step_body)

    gru_w = (p['conv_root'].astype(_F32), row(p['conv_bias']),
             p['gru_Wih'].T.astype(_F32), row(p['gru_bih']),
             p['gru_Whh'].T.astype(_F32), row(p['gru_bhh']))

    def gru_step(agg, h):
        return pl.pallas_call(
            _gru_body,
            grid=(GR,),
            in_specs=[
                pl.BlockSpec((RB, NEMB), lambda i: (i, 0)),
                pl.BlockSpec((RB, NP), lambda i: (i, 0)),
                *[w_spec(w) for w in gru_w],
            ],
            out_specs=pl.BlockSpec((RB, NP), lambda i: (i, 0)),
            out_shape=jax.ShapeDtypeStruct((N, NP), _F32),
        )(agg, h, *gru_w)

    h = out0
    for _ in range(NUM_CONV_STEPS):
        agg = sc_step(h, srcw, dstw, ab3)
        h = gru_step(agg, h)

    head_w = (p['stememb'].astype(_F32),
              p['s2p_W1'].T.astype(_F32), row(p['s2p_b1']),
              p['s2p_W2'].T.astype(_F32), row(p['s2p_b2']),
              p['s2p_W3'].T.astype(_F32), row(p['s2p_b3']),
              p['g2p_W1'].T.astype(_F32), row(p['g2p_b1']),
              p['g2p_W2'].T.astype(_F32), row(p['g2p_b2']))
    stem_preds, mol_preds = pl.pallas_call(
        _heads_body,
        grid=(G,),
        in_specs=[
            pl.BlockSpec((CN, NP), lambda i: (i, 0)),
            col_spec(CS), col_spec(CS),
            *[w_spec(w) for w in head_w],
        ],
        out_specs=[
            pl.BlockSpec((CS, OUT_PER_STEM), lambda i: (i, 0)),
            pl.BlockSpec((C, OUT_PER_MOL), lambda i: (i, 0)),
        ],
        out_shape=[
            jax.ShapeDtypeStruct((S, OUT_PER_STEM), _F32),
            jax.ShapeDtypeStruct((B, OUT_PER_MOL), _F32),
        ],
    )(h, stt3, stem03, *head_w)
    return stem_preds, mol_preds


def kernel(params, vec_data, x, stemtypes, edge_attr, edge_index, batch,
           stems_batch, stems, slices_x):
    return _run(params, vec_data, x, stemtypes, edge_attr, edge_index, stems)
